# baseline (speedup 1.0000x reference)
import functools

import jax
import jax.numpy as jnp
from jax import lax
from jax.experimental import pallas as pl
from jax.experimental.pallas import tpu as pltpu
from jax.experimental.pallas import tpu_sc as plsc

N = 10000
E = 320000
F = 128
H = 64
K = 10
ALPHA = 0.2

NC = 2
NS = 16
NW = NC * NS
CHUNK = 128
CPW = 80
EP = NW * CPW * CHUNK
NPAD = 10240
DR = NPAD // 16
DRT = DR // NS
RPT = NPAD // NS

_mesh = plsc.VectorSubcoreMesh(core_axis_name="c", subcore_axis_name="s")


@functools.partial(
    pl.kernel,
    out_type=jax.ShapeDtypeStruct((NC, NPAD), jnp.float32),
    mesh=_mesh,
    scratch_types=[
        pltpu.VMEM((CPW, CHUNK), jnp.int32),
        pltpu.VMEM((CPW, CHUNK), jnp.float32),
        pltpu.VMEM((RPT,), jnp.float32),
        pltpu.VMEM_SHARED((NPAD,), jnp.float32),
    ],
)
def _deg(col_hbm, w_hbm, out_hbm, col_v, w_v, zbuf, deg_sh):
    c = lax.axis_index("c")
    s = lax.axis_index("s")
    wid = s * NC + c
    base = wid * CPW
    pltpu.sync_copy(col_hbm.at[pl.ds(base, CPW)], col_v)
    pltpu.sync_copy(w_hbm.at[pl.ds(base, CPW)], w_v)

    zero16 = jnp.zeros((16,), jnp.float32)
    for i in range(RPT // 16):
        zbuf[pl.ds(i * 16, 16)] = zero16
    pltpu.sync_copy(zbuf, deg_sh.at[pl.ds(s * RPT, RPT)])
    plsc.subcore_barrier()

    def acc_body(ci, carry):
        pltpu.sync_copy(w_v.at[ci], deg_sh.at[col_v.at[ci]], add=True)
        return carry

    lax.fori_loop(0, CPW, acc_body, 0)

    plsc.subcore_barrier()
    pltpu.sync_copy(deg_sh.at[pl.ds(s * RPT, RPT)],
                    out_hbm.at[c].at[pl.ds(s * RPT, RPT)])


HC = H // NC
CPT = EP // (NS * CHUNK)


@functools.partial(
    pl.kernel,
    out_type=jax.ShapeDtypeStruct((NC, NPAD, HC), jnp.float32),
    mesh=_mesh,
    scratch_types=[
        pltpu.VMEM((CPT, CHUNK), jnp.int32),
        pltpu.VMEM((CPT, CHUNK), jnp.int32),
        pltpu.VMEM((CPT, CHUNK), jnp.float32),
        pltpu.VMEM((4, CHUNK, HC), jnp.float32),
        pltpu.VMEM((CHUNK, HC), jnp.float32),
        pltpu.VMEM_SHARED((NPAD, HC), jnp.float32),
        pltpu.VMEM_SHARED((NPAD, HC), jnp.float32),
        [pltpu.SemaphoreType.DMA] * 4,
        [pltpu.SemaphoreType.DMA] * 4,
    ],
    compiler_params=pltpu.CompilerParams(use_tc_tiling_on_sc=False),
)
def _edges(g_hbm, row_hbm, col_hbm, w_hbm, out_hbm,
           row_v, col_v, w_v, msg_v, zbuf, g_sh, acc_sh, gs, ss):
    c = lax.axis_index("c")
    s = lax.axis_index("s")
    base = s * CPT
    pltpu.sync_copy(row_hbm.at[pl.ds(base, CPT)], row_v)
    pltpu.sync_copy(col_hbm.at[pl.ds(base, CPT)], col_v)
    pltpu.sync_copy(w_hbm.at[pl.ds(base, CPT)], w_v)
    pltpu.sync_copy(g_hbm.at[c].at[pl.ds(s * RPT, RPT)],
                    g_sh.at[pl.ds(s * RPT, RPT)])

    zero16 = jnp.zeros((16,), jnp.float32)

    def zero_body(i, carry):
        for f in range(HC // 16):
            zbuf[i, pl.ds(f * 16, 16)] = zero16
        return carry

    lax.fori_loop(0, CHUNK, zero_body, 0)
    for t in range(RPT // CHUNK):
        pltpu.sync_copy(zbuf, acc_sh.at[pl.ds(s * RPT + t * CHUNK, CHUNK)])
    plsc.subcore_barrier()

    def scale(ci, b):
        for gi in range(CHUNK // 16):
            wv = w_v[ci, pl.ds(gi * 16, 16)]
            for j in range(16):
                e = gi * 16 + j
                sc = wv[j]
                for f in range(HC // 16):
                    msg_v[b, e, pl.ds(f * 16, 16)] = (
                        msg_v[b, e, pl.ds(f * 16, 16)] * sc)

    pltpu.async_copy(g_sh.at[row_v.at[0]], msg_v.at[0], gs[0])
    pltpu.async_copy(g_sh.at[row_v.at[1]], msg_v.at[1], gs[1])

    def edge_quad(q, carry):
        ci0 = q * 4
        for b in range(4):
            ci = ci0 + b
            b2 = (b + 2) % 4
            pltpu.make_async_copy(g_sh.at[row_v.at[ci]], msg_v.at[b],
                                  gs[b]).wait()
            scale(ci, b)
            pltpu.async_copy(msg_v.at[b], acc_sh.at[col_v.at[ci]], ss[b],
                             add=True)

            @pl.when(ci + 2 < CPT)
            def _():
                @pl.when(ci >= 2)
                def _():
                    pltpu.make_async_copy(msg_v.at[b2],
                                          acc_sh.at[col_v.at[ci - 2]],
                                          ss[b2]).wait()
                pltpu.async_copy(g_sh.at[row_v.at[ci + 2]], msg_v.at[b2],
                                 gs[b2])

        return carry

    lax.fori_loop(0, CPT // 4, edge_quad, 0)
    for b in range(4):
        pltpu.make_async_copy(msg_v.at[b], acc_sh.at[col_v.at[CPT - 4 + b]],
                              ss[b]).wait()

    plsc.subcore_barrier()
    pltpu.sync_copy(acc_sh.at[pl.ds(s * RPT, RPT)],
                    out_hbm.at[c].at[pl.ds(s * RPT, RPT)])


def _tc_h_body(x_ref, w_ref, h_ref):
    h_ref[...] = jnp.dot(x_ref[...], w_ref[...],
                         preferred_element_type=jnp.float32)


def _tc_g_body(h_ref, degp_ref, g_ref, dinv_ref):
    h = h_ref[...]
    deg = lax.dot_general(degp_ref[...], jnp.ones((NC, 1), jnp.float32),
                          (((0,), (0,)), ((), ())),
                          preferred_element_type=jnp.float32)[:N, :] + 1.0
    dinv = jnp.where(deg > 0.0, lax.rsqrt(deg), 0.0)
    g = dinv * h
    pad = jnp.zeros((NPAD - N, HC), jnp.float32)
    g_ref[0] = jnp.concatenate([g[:, :HC], pad], axis=0)
    g_ref[1] = jnp.concatenate([g[:, HC:], pad], axis=0)
    dinv_ref[...] = dinv


def _fin_body(accp_ref, g_ref, dinv_ref, b_ref, mu_ref, out_ref, q_ref):
    acc = jnp.concatenate([accp_ref[0, :N, :], accp_ref[1, :N, :]], axis=1)
    g = jnp.concatenate([g_ref[0, :N, :], g_ref[1, :N, :]], axis=1)
    dinv = dinv_ref[...]
    out = dinv * (acc + g) + b_ref[...][None, :]
    out_ref[...] = out
    mu = mu_ref[...]
    s_out = jnp.sum(out * out, axis=1, keepdims=True)
    om = lax.dot_general(out, mu, (((1,), (1,)), ((), ())),
                         preferred_element_type=jnp.float32)
    smu = lax.dot_general(jnp.ones((1, H), jnp.float32), mu * mu,
                          (((1,), (1,)), ((), ())),
                          preferred_element_type=jnp.float32)
    dist = s_out - 2.0 * om + smu
    qb = 1.0 / (1.0 + dist / ALPHA + 1e-8)
    q = qb ** (ALPHA + 1.0) / 2.0
    q_ref[...] = q / jnp.sum(q, axis=1, keepdims=True)


def kernel(x, edge_index, edge_attr, W, b, mu):
    row = edge_index[0]
    col = edge_index[1]
    pad = EP - E
    rowp = jnp.concatenate([row, jnp.zeros((pad,), row.dtype)])
    colp = jnp.concatenate([col, jnp.zeros((pad,), col.dtype)])
    wp = jnp.concatenate([edge_attr, jnp.zeros((pad,), edge_attr.dtype)])
    row2 = rowp.reshape(NW * CPW, CHUNK)
    col2 = colp.reshape(NW * CPW, CHUNK)
    w2 = wp.reshape(NW * CPW, CHUNK)

    degp = _deg(col2, w2)

    h = pl.pallas_call(
        _tc_h_body,
        out_shape=jax.ShapeDtypeStruct((N, H), jnp.float32),
    )(x, W)

    gsplit, dinv = pl.pallas_call(
        _tc_g_body,
        out_shape=[
            jax.ShapeDtypeStruct((NC, NPAD, HC), jnp.float32),
            jax.ShapeDtypeStruct((N, 1), jnp.float32),
        ],
    )(h, degp)

    accp = _edges(gsplit, row2, col2, w2)

    out, q = pl.pallas_call(
        _fin_body,
        out_shape=[
            jax.ShapeDtypeStruct((N, H), jnp.float32),
            jax.ShapeDtypeStruct((N, K), jnp.float32),
        ],
    )(accp, gsplit, dinv, b, mu)
    return (out, q)


# speedup vs baseline: 1.0025x; 1.0025x over previous
import functools

import jax
import jax.numpy as jnp
from jax import lax
from jax.experimental import pallas as pl
from jax.experimental.pallas import tpu as pltpu
from jax.experimental.pallas import tpu_sc as plsc

N = 10000
E = 320000
F = 128
H = 64
K = 10
ALPHA = 0.2

NC = 2
NS = 16
NW = NC * NS
CHUNK = 128
CPW = 80
EP = NW * CPW * CHUNK
NPAD = 10240
DR = NPAD // 16
DRT = DR // NS
RPT = NPAD // NS

_mesh = plsc.VectorSubcoreMesh(core_axis_name="c", subcore_axis_name="s")


@functools.partial(
    pl.kernel,
    out_type=jax.ShapeDtypeStruct((NC, NPAD), jnp.float32),
    mesh=_mesh,
    scratch_types=[
        pltpu.VMEM((CPW, CHUNK), jnp.int32),
        pltpu.VMEM((CPW, CHUNK), jnp.float32),
        pltpu.VMEM((RPT,), jnp.float32),
        pltpu.VMEM_SHARED((NPAD,), jnp.float32),
    ],
)
def _deg(col_hbm, w_hbm, out_hbm, col_v, w_v, zbuf, deg_sh):
    c = lax.axis_index("c")
    s = lax.axis_index("s")
    wid = s * NC + c
    base = wid * CPW
    pltpu.sync_copy(col_hbm.at[pl.ds(base, CPW)], col_v)
    pltpu.sync_copy(w_hbm.at[pl.ds(base, CPW)], w_v)

    zero16 = jnp.zeros((16,), jnp.float32)
    for i in range(RPT // 16):
        zbuf[pl.ds(i * 16, 16)] = zero16
    pltpu.sync_copy(zbuf, deg_sh.at[pl.ds(s * RPT, RPT)])
    plsc.subcore_barrier()

    def acc_body(ci, carry):
        pltpu.sync_copy(w_v.at[ci], deg_sh.at[col_v.at[ci]], add=True)
        return carry

    lax.fori_loop(0, CPW, acc_body, 0)

    plsc.subcore_barrier()
    pltpu.sync_copy(deg_sh.at[pl.ds(s * RPT, RPT)],
                    out_hbm.at[c].at[pl.ds(s * RPT, RPT)])


HC = H // NC
CPT = EP // (NS * CHUNK)


@functools.partial(
    pl.kernel,
    out_type=jax.ShapeDtypeStruct((NC, NPAD, HC), jnp.float32),
    mesh=_mesh,
    scratch_types=[
        pltpu.VMEM((CPT, CHUNK), jnp.int32),
        pltpu.VMEM((CPT, CHUNK), jnp.int32),
        pltpu.VMEM((CPT, CHUNK), jnp.float32),
        pltpu.VMEM((4, CHUNK, HC), jnp.float32),
        pltpu.VMEM((CHUNK, HC), jnp.float32),
        pltpu.VMEM_SHARED((NPAD, HC), jnp.float32),
        pltpu.VMEM_SHARED((NPAD, HC), jnp.float32),
        [pltpu.SemaphoreType.DMA] * 4,
        [pltpu.SemaphoreType.DMA] * 4,
    ],
    compiler_params=pltpu.CompilerParams(use_tc_tiling_on_sc=False),
)
def _edges(g_hbm, row_hbm, col_hbm, w_hbm, out_hbm,
           row_v, col_v, w_v, msg_v, zbuf, g_sh, acc_sh, gs, ss):
    c = lax.axis_index("c")
    s = lax.axis_index("s")
    base = s * CPT
    pltpu.sync_copy(row_hbm.at[pl.ds(base, CPT)], row_v)
    pltpu.sync_copy(col_hbm.at[pl.ds(base, CPT)], col_v)
    pltpu.sync_copy(w_hbm.at[pl.ds(base, CPT)], w_v)
    pltpu.sync_copy(g_hbm.at[c].at[pl.ds(s * RPT, RPT)],
                    g_sh.at[pl.ds(s * RPT, RPT)])

    zero16 = jnp.zeros((16,), jnp.float32)

    def zero_body(i, carry):
        for f in range(HC // 16):
            zbuf[i, pl.ds(f * 16, 16)] = zero16
        return carry

    lax.fori_loop(0, CHUNK, zero_body, 0)
    for t in range(RPT // CHUNK):
        pltpu.sync_copy(zbuf, acc_sh.at[pl.ds(s * RPT + t * CHUNK, CHUNK)])
    plsc.subcore_barrier()

    def scale(ci, b):
        for gi in range(CHUNK // 16):
            wv = w_v[ci, pl.ds(gi * 16, 16)]
            for j in range(16):
                e = gi * 16 + j
                sc = wv[j]
                for f in range(HC // 16):
                    msg_v[b, e, pl.ds(f * 16, 16)] = (
                        msg_v[b, e, pl.ds(f * 16, 16)] * sc)

    pltpu.async_copy(g_sh.at[row_v.at[0]], msg_v.at[0], gs[0])
    pltpu.async_copy(g_sh.at[row_v.at[1]], msg_v.at[1], gs[1])

    def edge_quad(q, carry):
        ci0 = q * 4
        for b in range(4):
            ci = ci0 + b
            b2 = (b + 2) % 4
            pltpu.make_async_copy(g_sh.at[row_v.at[ci]], msg_v.at[b],
                                  gs[b]).wait()
            scale(ci, b)
            pltpu.async_copy(msg_v.at[b], acc_sh.at[col_v.at[ci]], ss[b],
                             add=True)

            @pl.when(ci + 2 < CPT)
            def _():
                @pl.when(ci >= 2)
                def _():
                    pltpu.make_async_copy(msg_v.at[b2],
                                          acc_sh.at[col_v.at[ci - 2]],
                                          ss[b2]).wait()
                pltpu.async_copy(g_sh.at[row_v.at[ci + 2]], msg_v.at[b2],
                                 gs[b2])

        return carry

    lax.fori_loop(0, CPT // 4, edge_quad, 0)
    for b in range(4):
        pltpu.make_async_copy(msg_v.at[b], acc_sh.at[col_v.at[CPT - 4 + b]],
                              ss[b]).wait()

    plsc.subcore_barrier()
    pltpu.sync_copy(acc_sh.at[pl.ds(s * RPT, RPT)],
                    out_hbm.at[c].at[pl.ds(s * RPT, RPT)])


def _tc_g_body(x_ref, w_ref, degp_ref, g_ref, dinv_ref):
    h = jnp.dot(x_ref[...], w_ref[...], preferred_element_type=jnp.float32)
    deg = lax.dot_general(degp_ref[...], jnp.ones((NC, 1), jnp.float32),
                          (((0,), (0,)), ((), ())),
                          preferred_element_type=jnp.float32)[:N, :] + 1.0
    dinv = jnp.where(deg > 0.0, lax.rsqrt(deg), 0.0)
    g = dinv * h
    pad = jnp.zeros((NPAD - N, HC), jnp.float32)
    g_ref[0] = jnp.concatenate([g[:, :HC], pad], axis=0)
    g_ref[1] = jnp.concatenate([g[:, HC:], pad], axis=0)
    dinv_ref[...] = dinv


def _fin_body(accp_ref, g_ref, dinv_ref, b_ref, mu_ref, out_ref, q_ref):
    acc = jnp.concatenate([accp_ref[0, :N, :], accp_ref[1, :N, :]], axis=1)
    g = jnp.concatenate([g_ref[0, :N, :], g_ref[1, :N, :]], axis=1)
    dinv = dinv_ref[...]
    out = dinv * (acc + g) + b_ref[...][None, :]
    out_ref[...] = out
    mu = mu_ref[...]
    s_out = jnp.sum(out * out, axis=1, keepdims=True)
    om = lax.dot_general(out, mu, (((1,), (1,)), ((), ())),
                         preferred_element_type=jnp.float32)
    smu = lax.dot_general(jnp.ones((1, H), jnp.float32), mu * mu,
                          (((1,), (1,)), ((), ())),
                          preferred_element_type=jnp.float32)
    dist = s_out - 2.0 * om + smu
    qb = 1.0 / (1.0 + dist / ALPHA + 1e-8)
    q = qb ** (ALPHA + 1.0) / 2.0
    q_ref[...] = q / jnp.sum(q, axis=1, keepdims=True)


def kernel(x, edge_index, edge_attr, W, b, mu):
    row = edge_index[0]
    col = edge_index[1]
    pad = EP - E
    rowp = jnp.concatenate([row, jnp.zeros((pad,), row.dtype)])
    colp = jnp.concatenate([col, jnp.zeros((pad,), col.dtype)])
    wp = jnp.concatenate([edge_attr, jnp.zeros((pad,), edge_attr.dtype)])
    row2 = rowp.reshape(NW * CPW, CHUNK)
    col2 = colp.reshape(NW * CPW, CHUNK)
    w2 = wp.reshape(NW * CPW, CHUNK)

    degp = _deg(col2, w2)

    gsplit, dinv = pl.pallas_call(
        _tc_g_body,
        out_shape=[
            jax.ShapeDtypeStruct((NC, NPAD, HC), jnp.float32),
            jax.ShapeDtypeStruct((N, 1), jnp.float32),
        ],
    )(x, W, degp)

    accp = _edges(gsplit, row2, col2, w2)

    out, q = pl.pallas_call(
        _fin_body,
        out_shape=[
            jax.ShapeDtypeStruct((N, H), jnp.float32),
            jax.ShapeDtypeStruct((N, K), jnp.float32),
        ],
    )(accp, gsplit, dinv, b, mu)
    return (out, q)
